# baseline (speedup 1.0000x reference)
import functools

import jax
import jax.numpy as jnp
from jax import lax
from jax.experimental import pallas as pl
from jax.experimental.pallas import tpu as pltpu
from jax.experimental.pallas import tpu_sc as plsc

_NUM_CORES = 2
_NUM_SUBCORES = 16
_NUM_WORKERS = _NUM_CORES * _NUM_SUBCORES

_NSLOT = 5
_LEAD = 4


def _gather_kernel(idx_hbm, table_hbm, out_hbm, idx_v, bufs, gsems, ssems, *,
                   seq, bpw):
    wid = lax.axis_index("s") * _NUM_CORES + lax.axis_index("c")
    col0 = wid * bpw
    pltpu.sync_copy(idx_hbm.at[:, pl.ds(col0, bpw)], idx_v)

    def g_start(j, b):
        pltpu.async_copy(table_hbm.at[idx_v.at[j]], bufs[b], gsems[b])

    def g_wait(j, b):
        pltpu.make_async_copy(table_hbm.at[idx_v.at[j]], bufs[b],
                              gsems[b]).wait()

    def s_start(j, b):
        pltpu.async_copy(bufs[b], out_hbm.at[j, pl.ds(col0, bpw)], ssems[b])

    def s_wait(b):
        pltpu.make_async_copy(bufs[b], out_hbm.at[0, pl.ds(col0, bpw)],
                              ssems[b]).wait()

    for p in range(_LEAD):
        g_start(p, p)

    for j in range(_NSLOT):
        g_wait(j, j)
        s_start(j, j)
        jn = j + _LEAD
        bn = jn % _NSLOT
        if jn >= _NSLOT:
            s_wait(bn)
        g_start(jn, bn)

    def step(i, carry):
        for b in range(_NSLOT):
            j = i * _NSLOT + b
            g_wait(j, b)
            s_start(j, b)
            jn = j + _LEAD
            bn = (b + _LEAD) % _NSLOT

            @pl.when(jn < seq)
            def _():
                s_wait(bn)
                g_start(jn, bn)

        return carry

    lax.fori_loop(1, seq // _NSLOT, step, 0)

    for b in range(_NSLOT):
        s_wait(b)


def kernel(indices, weight):
    b0, seq = indices.shape
    vocab, dim = weight.shape
    assert b0 % _NUM_WORKERS == 0
    bpw = b0 // _NUM_WORKERS
    assert seq % _NSLOT == 0

    idx_t = indices.astype(jnp.int32).T

    mesh = plsc.VectorSubcoreMesh(core_axis_name="c", subcore_axis_name="s",
                                  num_cores=_NUM_CORES,
                                  num_subcores=_NUM_SUBCORES)
    grid_kernel = pl.kernel(
        functools.partial(_gather_kernel, seq=seq, bpw=bpw),
        out_type=jax.ShapeDtypeStruct((seq, b0, dim), jnp.float32),
        mesh=mesh,
        scratch_types=[
            pltpu.VMEM((seq, bpw), jnp.int32),
            [pltpu.VMEM((bpw, dim), jnp.float32) for _ in range(_NSLOT)],
            [pltpu.SemaphoreType.DMA for _ in range(_NSLOT)],
            [pltpu.SemaphoreType.DMA for _ in range(_NSLOT)],
        ],
    )
    out = grid_kernel(idx_t, weight)
    return out.transpose(1, 0, 2)


# speedup vs baseline: 1.0037x; 1.0037x over previous
import functools

import jax
import jax.numpy as jnp
from jax import lax
from jax.experimental import pallas as pl
from jax.experimental.pallas import tpu as pltpu
from jax.experimental.pallas import tpu_sc as plsc

_NUM_CORES = 2
_NUM_SUBCORES = 16
_NUM_WORKERS = _NUM_CORES * _NUM_SUBCORES

_NSLOT = 5
_LEAD = 3


def _gather_kernel(idx_hbm, table_hbm, out_hbm, idx_v, bufs, gsems, ssems, *,
                   seq, bpw):
    wid = lax.axis_index("s") * _NUM_CORES + lax.axis_index("c")
    col0 = wid * bpw
    pltpu.sync_copy(idx_hbm.at[:, pl.ds(col0, bpw)], idx_v)

    def g_start(j, b):
        pltpu.async_copy(table_hbm.at[idx_v.at[j]], bufs[b], gsems[b])

    def g_wait(j, b):
        pltpu.make_async_copy(table_hbm.at[idx_v.at[j]], bufs[b],
                              gsems[b]).wait()

    def s_start(j, b):
        pltpu.async_copy(bufs[b], out_hbm.at[j, pl.ds(col0, bpw)], ssems[b])

    def s_wait(b):
        pltpu.make_async_copy(bufs[b], out_hbm.at[0, pl.ds(col0, bpw)],
                              ssems[b]).wait()

    for p in range(_LEAD):
        g_start(p, p)

    for j in range(_NSLOT):
        g_wait(j, j)
        s_start(j, j)
        jn = j + _LEAD
        bn = jn % _NSLOT
        if jn >= _NSLOT:
            s_wait(bn)
        g_start(jn, bn)

    def step(i, carry):
        for b in range(_NSLOT):
            j = i * _NSLOT + b
            g_wait(j, b)
            s_start(j, b)
            jn = j + _LEAD
            bn = (b + _LEAD) % _NSLOT

            @pl.when(jn < seq)
            def _():
                s_wait(bn)
                g_start(jn, bn)

        return carry

    lax.fori_loop(1, seq // _NSLOT, step, 0)

    for b in range(_NSLOT):
        s_wait(b)


def kernel(indices, weight):
    b0, seq = indices.shape
    vocab, dim = weight.shape
    assert b0 % _NUM_WORKERS == 0
    bpw = b0 // _NUM_WORKERS
    assert seq % _NSLOT == 0

    idx_t = indices.astype(jnp.int32).T

    mesh = plsc.VectorSubcoreMesh(core_axis_name="c", subcore_axis_name="s",
                                  num_cores=_NUM_CORES,
                                  num_subcores=_NUM_SUBCORES)
    grid_kernel = pl.kernel(
        functools.partial(_gather_kernel, seq=seq, bpw=bpw),
        out_type=jax.ShapeDtypeStruct((seq, b0, dim), jnp.float32),
        mesh=mesh,
        scratch_types=[
            pltpu.VMEM((seq, bpw), jnp.int32),
            [pltpu.VMEM((bpw, dim), jnp.float32) for _ in range(_NSLOT)],
            [pltpu.SemaphoreType.DMA for _ in range(_NSLOT)],
            [pltpu.SemaphoreType.DMA for _ in range(_NSLOT)],
        ],
    )
    out = grid_kernel(idx_t, weight)
    return out.transpose(1, 0, 2)
